# baseline (speedup 1.0000x reference)
import jax
import jax.numpy as jnp
from jax.experimental import pallas as pl

_N = 700
_HEADS = 8
_HID = 8
_XD = 128
_LOG2E = 1.4426950408889634


def _leaky(v):
    return jnp.maximum(v, 0.2 * v)


def _attend(g_aug, col1, col2, as_row):
    row2 = 0.2 * as_row
    p = jnp.exp2(jnp.maximum(col1 + as_row, col2 + row2))
    return jnp.dot(p, g_aug, preferred_element_type=jnp.float32)


def _bd_map(a_ref):
    r = jax.lax.broadcasted_iota(jnp.int32, (_HEADS * _HID, _HID), 0)
    k = jax.lax.broadcasted_iota(jnp.int32, (_HEADS * _HID, _HID), 1)
    sel_head = (r // _HID == k).astype(jnp.float32)
    sel_ch = (r % _HID == k).astype(jnp.float32)
    rows = jnp.dot(sel_head, _LOG2E * a_ref[...],
                   preferred_element_type=jnp.float32)
    vals = jnp.sum(rows * sel_ch, axis=1, keepdims=True)
    return sel_head * vals


def _gat_body(x_ref, w1_ref, as1_ref, ad1_ref, b1_ref,
              w2_ref, as2_ref, ad2_ref, b2_ref, o_ref):
    ones_col = jnp.ones((_N, 1), jnp.float32)

    h = jnp.dot(x_ref[...], w1_ref[...], preferred_element_type=jnp.float32)
    h_aug = jnp.concatenate([h, ones_col], axis=1)
    ad_all = jnp.dot(h, _bd_map(ad1_ref),
                     preferred_element_type=jnp.float32)
    as_all = jnp.dot(h, _bd_map(as1_ref),
                     preferred_element_type=jnp.float32)
    as_rows = jnp.transpose(as_all)
    as_maxs = jnp.max(as_all, axis=0, keepdims=True)
    c_all = _leaky(ad_all + as_maxs)
    col1_all = ad_all - c_all
    col2_all = 0.2 * ad_all - c_all

    pieces = []
    for i in range(_HEADS):
        aug = _attend(h_aug, col1_all[:, i:i + 1],
                      col2_all[:, i:i + 1], as_rows[i:i + 1, :])
        num = aug[:, i * _HID:(i + 1) * _HID]
        den = aug[:, _HEADS * _HID:]
        pieces.append(num * (1.0 / (den + 1e-16)))
    h1 = jnp.concatenate(pieces, axis=1) + b1_ref[...]
    h1 = jnp.maximum(h1, 0.0)

    g = jnp.dot(h1, w2_ref[...], preferred_element_type=jnp.float32)
    g_aug = jnp.concatenate([g, ones_col], axis=1)
    ad2 = jnp.dot(g, jnp.transpose(_LOG2E * ad2_ref[...]),
                  preferred_element_type=jnp.float32)
    as2 = jnp.dot(g, jnp.transpose(_LOG2E * as2_ref[...]),
                  preferred_element_type=jnp.float32)
    as2_row = jnp.transpose(as2)
    as2_max = jnp.max(as2, axis=0, keepdims=True)
    c2 = _leaky(ad2 + as2_max)
    aug = _attend(g_aug, ad2 - c2, 0.2 * ad2 - c2, as2_row)
    o_ref[...] = (aug[:, :_XD] * (1.0 / (aug[:, _XD:] + 1e-16))
                  + b2_ref[...])


def kernel(x, W1, a_src1, a_dst1, b1, W2, a_src2, a_dst2, b2):
    return pl.pallas_call(
        _gat_body,
        out_shape=jax.ShapeDtypeStruct((_N, _XD), jnp.float32),
    )(x, W1, a_src1, a_dst1, b1.reshape(1, -1),
      W2, a_src2, a_dst2, b2.reshape(1, -1))


# speedup vs baseline: 1.0157x; 1.0157x over previous
import jax
import jax.numpy as jnp
from jax.experimental import pallas as pl

_N = 700
_HEADS = 8
_HID = 8
_XD = 128
_LOG2E = 1.4426950408889634


def _leaky(v):
    return jnp.maximum(v, 0.2 * v)


def _attend(g_aug, ad_col, as_row, as_max):
    c = _leaky(ad_col + as_max)
    col1 = ad_col - c
    col2 = 0.2 * ad_col - c
    row2 = 0.2 * as_row
    p = jnp.exp2(jnp.maximum(col1 + as_row, col2 + row2))
    return jnp.dot(p, g_aug, preferred_element_type=jnp.float32)


def _bd_map(a_ref):
    r = jax.lax.broadcasted_iota(jnp.int32, (_HEADS * _HID, _HID), 0)
    k = jax.lax.broadcasted_iota(jnp.int32, (_HEADS * _HID, _HID), 1)
    sel_head = (r // _HID == k).astype(jnp.float32)
    sel_ch = (r % _HID == k).astype(jnp.float32)
    rows = jnp.dot(sel_head, _LOG2E * a_ref[...],
                   preferred_element_type=jnp.float32)
    vals = jnp.sum(rows * sel_ch, axis=1, keepdims=True)
    return sel_head * vals


def _gat_body(x_ref, w1_ref, as1_ref, ad1_ref, b1_ref,
              w2_ref, as2_ref, ad2_ref, b2_ref, o_ref):
    ones_col = jnp.ones((_N, 1), jnp.float32)

    h = jnp.dot(x_ref[...], w1_ref[...], preferred_element_type=jnp.float32)
    h_aug = jnp.concatenate([h, ones_col], axis=1)
    ad_all = jnp.dot(h, _bd_map(ad1_ref),
                     preferred_element_type=jnp.float32)
    as_all = jnp.dot(h, _bd_map(as1_ref),
                     preferred_element_type=jnp.float32)
    as_rows = jnp.transpose(as_all)
    as_maxs = jnp.max(as_rows, axis=1, keepdims=True)

    pieces = []
    for i in range(_HEADS):
        aug = _attend(h_aug, ad_all[:, i:i + 1],
                      as_rows[i:i + 1, :], as_maxs[i:i + 1, :])
        num = aug[:, i * _HID:(i + 1) * _HID]
        den = aug[:, _HEADS * _HID:]
        pieces.append(num * (1.0 / (den + 1e-16)))
    h1 = jnp.concatenate(pieces, axis=1) + b1_ref[...]
    h1 = jnp.maximum(h1, 0.0)

    g = jnp.dot(h1, w2_ref[...], preferred_element_type=jnp.float32)
    g_aug = jnp.concatenate([g, ones_col], axis=1)
    ad2 = jnp.dot(g, jnp.transpose(_LOG2E * ad2_ref[...]),
                  preferred_element_type=jnp.float32)
    as2 = jnp.dot(g, jnp.transpose(_LOG2E * as2_ref[...]),
                  preferred_element_type=jnp.float32)
    as2_row = jnp.transpose(as2)
    as2_max = jnp.max(as2_row, axis=1, keepdims=True)
    aug = _attend(g_aug, ad2, as2_row, as2_max)
    o_ref[...] = (aug[:, :_XD] * (1.0 / (aug[:, _XD:] + 1e-16))
                  + b2_ref[...])


def kernel(x, W1, a_src1, a_dst1, b1, W2, a_src2, a_dst2, b2):
    return pl.pallas_call(
        _gat_body,
        out_shape=jax.ShapeDtypeStruct((_N, _XD), jnp.float32),
    )(x, W1, a_src1, a_dst1, b1.reshape(1, -1),
      W2, a_src2, a_dst2, b2.reshape(1, -1))
